# Initial kernel scaffold; baseline (speedup 1.0000x reference)
import jax
import jax.numpy as jnp
from jax.experimental import pallas as pl


def kernel(vertices, params):
    raise NotImplementedError("write your pallas kernel here")


# speedup vs baseline: 1.0014x; 1.0014x over previous
import functools

import jax
import jax.numpy as jnp
from jax import lax
from jax.experimental import pallas as pl
from jax.experimental.pallas import tpu as pltpu

_SUPPORT = 1
_NB_L = 10
_NB_G = 50


def _knn_body(K, pos_ref, out_ref, d_ref):
    pos = pos_ref[0]
    v = pos.shape[0]
    inner = jnp.dot(pos, pos.T, preferred_element_type=jnp.float32)
    sq = jnp.sum(pos * pos, axis=1)
    d_ref[...] = (sq[:, None] - 2.0 * inner) + sq[None, :]
    iota = lax.broadcasted_iota(jnp.int32, (v, v), 1)

    def step(j, _):
        d = d_ref[...]
        m = jnp.min(d, axis=1, keepdims=True)
        eq = d == m
        amin = jnp.min(jnp.where(eq, iota, v), axis=1)
        out_ref[0, pl.ds(j, 1), :] = amin[None, :]
        d_ref[...] = jnp.where(iota == amin[:, None], jnp.inf, d)
        return 0

    lax.fori_loop(0, K, step, 0)


def _knn_topk(pos, K):
    B, V, _ = pos.shape
    out = pl.pallas_call(
        functools.partial(_knn_body, K),
        grid=(B,),
        in_specs=[pl.BlockSpec((1, V, 3), lambda b: (b, 0, 0))],
        out_specs=pl.BlockSpec((1, K, V), lambda b: (b, 0, 0)),
        out_shape=jax.ShapeDtypeStruct((B, K, V), jnp.int32),
        scratch_shapes=[pltpu.VMEM((V, V), jnp.float32)],
    )(pos)
    return jnp.transpose(out, (0, 2, 1))


def _gather_nb(t, idx):
    return jax.vmap(lambda tb, ib: tb[ib])(t, idx)


def _nb_dir_norm(vertices, idx):
    nb = _gather_nb(vertices, idx)
    d = nb - vertices[:, :, None, :]
    n = jnp.linalg.norm(d, axis=-1, keepdims=True)
    return d / jnp.maximum(n, 1e-12)


def _conv_surface(p, idx, vertices, kernel_num, s):
    bs, v, n = idx.shape
    ndn = _nb_dir_norm(vertices, idx)
    dnorm = jnp.maximum(jnp.linalg.norm(p['dir'], axis=0, keepdims=True), 1e-12)
    sdn = p['dir'] / dnorm
    theta = jax.nn.relu(ndn @ sdn)
    theta = theta.reshape(bs, v, n, s, kernel_num)
    theta = jnp.max(theta, axis=2)
    return jnp.sum(theta, axis=2)


def _conv_layer(p, idx, vertices, fmap, out_ch, s):
    bs, v, n = idx.shape
    ndn = _nb_dir_norm(vertices, idx)
    dnorm = jnp.maximum(jnp.linalg.norm(p['dir'], axis=0, keepdims=True), 1e-12)
    sdn = p['dir'] / dnorm
    theta = jax.nn.relu(ndn @ sdn)
    fout = fmap @ p['w'] + p['b']
    fc = fout[:, :, :out_ch]
    fs = fout[:, :, out_ch:]
    fs_nb = _gather_nb(fs, idx)
    act = (theta * fs_nb).reshape(bs, v, n, s, out_ch)
    act = jnp.max(act, axis=2)
    act = jnp.sum(act, axis=2)
    return fc + act


def _bn(p, x):
    mu = jnp.mean(x, axis=(0, 1), keepdims=True)
    var = jnp.var(x, axis=(0, 1), keepdims=True)
    return (x - mu) / jnp.sqrt(var + 1e-5) * p['g'] + p['b']


def _fusion_surface(p, vertices, knn, dim, s):
    il = knn[:, :, 1:_NB_L + 1]
    ig = knn[:, :, 1:_NB_G + 1]
    fl = jax.nn.relu(_bn(p['bn_l'], _conv_surface(p['conv_l'], il, vertices, dim, s)))
    fg = jax.nn.relu(_bn(p['bn_g0'], _conv_surface(p['conv_g0'], ig, vertices, dim, s)))
    fg = jax.nn.relu(_bn(p['bn_g1'], _conv_layer(p['conv_g1'], ig, vertices, fg, dim, s)))
    return jnp.concatenate([fl, fg], axis=2)


def _fusion(p, vertices, knn, inp, dim, s):
    il = knn[:, :, 1:_NB_L + 1]
    ig = knn[:, :, 1:_NB_G + 1]
    fl = jax.nn.relu(_bn(p['bn_l'], _conv_layer(p['conv_l'], il, vertices, inp, dim, s)))
    fg = jax.nn.relu(_bn(p['bn_g0'], _conv_layer(p['conv_g0'], ig, vertices, inp, dim, s)))
    fg = jax.nn.relu(_bn(p['bn_g1'], _conv_layer(p['conv_g1'], ig, vertices, fg, dim, s)))
    return jnp.concatenate([fl, fg], axis=2)


def _linear_relu(p, x):
    return jax.nn.relu(x @ p['w'] + p['b'])


def _pool(vertices, fmap, knn, rate=4, nbk=4):
    idx = knn[:, :, 1:nbk + 1]
    nb = _gather_nb(fmap, idx)
    pooled = jnp.max(nb, axis=2)
    pool_num = vertices.shape[1] // rate
    return vertices[:, :pool_num, :], pooled[:, :pool_num, :]


def _transformer(p, pos, feat, knn, n_knn=16):
    identity = feat
    x = feat @ p['start']['w'] + p['start']['b']
    q = x @ p['q']['w'] + p['q']['b']
    k = x @ p['k']['w'] + p['k']['b']
    v = x @ p['v']['w'] + p['v']['b']
    idx = knn[:, :, 1:n_knn + 1]
    knb = _gather_nb(k, idx)
    vnb = _gather_nb(v, idx)
    pnb = _gather_nb(pos, idx)
    rel = pos[:, :, None, :] - pnb
    pe = jax.nn.relu(rel @ p['pos1']['w'] + p['pos1']['b']) @ p['pos2']['w'] + p['pos2']['b']
    a = jax.nn.relu((q[:, :, None, :] - knb + pe) @ p['attn1']['w'] + p['attn1']['b']) @ p['attn2']['w'] + p['attn2']['b']
    a = jax.nn.softmax(a, axis=2)
    agg = jnp.sum(a * (vnb + pe), axis=2)
    return agg @ p['end']['w'] + p['end']['b'] + identity


def kernel(vertices, params):
    S = _SUPPORT
    v = jnp.transpose(vertices, (0, 2, 1))

    knn0 = _knn_topk(v, 51)
    fm0 = _fusion_surface(params['conv_0'], v, knn0, 128, S)
    fm0 = _linear_relu(params['down0'], fm0)
    fm0 = _transformer(params['att0'], v, fm0, knn0)
    fm1 = _fusion(params['conv_1'], v, knn0, fm0, 128, S)
    fm1 = _linear_relu(params['down1'], fm1)
    fm1 = _transformer(params['att1'], v, fm1, knn0)
    vp1, fp1 = _pool(v, fm1, knn0)

    knn1 = _knn_topk(vp1, 51)
    fm2 = _fusion(params['conv_2'], vp1, knn1, fp1, 128, S)
    fm2 = _transformer(params['att2'], vp1, fm2, knn1)
    fm3 = _fusion(params['conv_3'], vp1, knn1, fm2, 256, S)
    fm3 = _transformer(params['att3'], vp1, fm3, knn1)
    vp2, fp2 = _pool(vp1, fm3, knn1)

    knn2 = _knn_topk(vp2, 51)
    fm4 = _fusion(params['conv_4'], vp2, knn2, fp2, 512, S)
    fm4 = _linear_relu(params['down2'], fm4)
    fm4 = _transformer(params['att4'], vp2, fm4, knn2)
    return jnp.max(fm4, axis=1)


# speedup vs baseline: 2.9669x; 2.9628x over previous
import functools

import jax
import jax.numpy as jnp
from jax import lax
from jax.experimental import pallas as pl
from jax.experimental.pallas import tpu as pltpu
from jax.experimental.pallas import tpu_sc as plsc

_SUPPORT = 1
_NB_L = 10
_NB_G = 50


def _knn_body(K, dist_ref, out_ref, d_ref):
    d_ref[...] = dist_ref[0]
    v = d_ref.shape[0]
    iota = lax.broadcasted_iota(jnp.int32, (v, v), 1)

    def step(j, _):
        d = d_ref[...]
        m = jnp.min(d, axis=1, keepdims=True)
        eq = d == m
        amin = jnp.min(jnp.where(eq, iota, v), axis=1)
        out_ref[0, pl.ds(j, 1), :] = amin[None, :]
        d_ref[...] = jnp.where(iota == amin[:, None], jnp.inf, d)
        return 0

    lax.fori_loop(0, K, step, 0)


def _knn_topk(pos, K):
    B, V, _ = pos.shape
    inner = jnp.einsum('bnd,bmd->bnm', pos, pos)
    sq = jnp.sum(pos * pos, axis=-1)
    dist = sq[:, :, None] - 2.0 * inner + sq[:, None, :]
    out = pl.pallas_call(
        functools.partial(_knn_body, K),
        grid=(B,),
        in_specs=[pl.BlockSpec((1, V, V), lambda b: (b, 0, 0))],
        out_specs=pl.BlockSpec((1, K, V), lambda b: (b, 0, 0)),
        out_shape=jax.ShapeDtypeStruct((B, K, V), jnp.int32),
        scratch_shapes=[pltpu.VMEM((V, V), jnp.float32)],
    )(dist)
    return jnp.transpose(out, (0, 2, 1))


_NW = 32


def _pick_chunk(n_per_w):
    for c in (128, 120, 112, 104, 96, 88, 80, 72, 64, 56, 48, 40, 32, 24, 16, 8):
        if c <= n_per_w and n_per_w % c == 0:
            return c
    raise ValueError(n_per_w)


@functools.cache
def _sc_gather_fn(N, R, D):
    n_per_w = N // _NW
    chunk = _pick_chunk(n_per_w)
    n_chunks = n_per_w // chunk
    mesh = plsc.VectorSubcoreMesh(core_axis_name="c", subcore_axis_name="s")

    @functools.partial(
        pl.kernel,
        mesh=mesh,
        out_type=jax.ShapeDtypeStruct((N, D), jnp.float32),
        scratch_types=[
            pltpu.VMEM((chunk,), jnp.int32),
            pltpu.VMEM((chunk, D), jnp.float32),
            pltpu.SemaphoreType.DMA,
        ],
        compiler_params=pltpu.CompilerParams(use_tc_tiling_on_sc=False),
    )
    def gather_k(table_hbm, idx_hbm, out_hbm, idx_v, rows_v, sem):
        wid = lax.axis_index("s") * 2 + lax.axis_index("c")
        base = wid * n_per_w

        def step(i, _):
            off = base + i * chunk
            pltpu.sync_copy(idx_hbm.at[pl.ds(off, chunk)], idx_v)
            pltpu.async_copy(table_hbm.at[idx_v], rows_v, sem).wait()
            pltpu.sync_copy(rows_v, out_hbm.at[pl.ds(off, chunk)])
            return 0

        lax.fori_loop(0, n_chunks, step, 0)

    return gather_k


def _gather_nb(t, idx):
    B, V, C = t.shape
    _, V2, k = idx.shape
    Cp = C if C % 16 == 0 else (C + 15) // 16 * 16
    tp = t if Cp == C else jnp.pad(t, ((0, 0), (0, 0), (0, Cp - C)))
    gidx = (idx + (jnp.arange(B, dtype=idx.dtype) * V)[:, None, None]).reshape(-1)
    rows = _sc_gather_fn(gidx.shape[0], B * V, Cp)(tp.reshape(B * V, Cp), gidx)
    rows = rows.reshape(B, V2, k, Cp)
    return rows if Cp == C else rows[..., :C]


def _gather_nb_tc(t, idx):
    return jax.vmap(lambda tb, ib: tb[ib])(t, idx)


def _nb_dir_norm(vertices, idx):
    nb = _gather_nb_tc(vertices, idx)
    d = nb - vertices[:, :, None, :]
    n = jnp.linalg.norm(d, axis=-1, keepdims=True)
    return d / jnp.maximum(n, 1e-12)


def _conv_surface(p, idx, vertices, kernel_num, s):
    bs, v, n = idx.shape
    ndn = _nb_dir_norm(vertices, idx)
    dnorm = jnp.maximum(jnp.linalg.norm(p['dir'], axis=0, keepdims=True), 1e-12)
    sdn = p['dir'] / dnorm
    theta = jax.nn.relu(ndn @ sdn)
    theta = theta.reshape(bs, v, n, s, kernel_num)
    theta = jnp.max(theta, axis=2)
    return jnp.sum(theta, axis=2)


def _conv_layer(p, idx, vertices, fmap, out_ch, s):
    bs, v, n = idx.shape
    ndn = _nb_dir_norm(vertices, idx)
    dnorm = jnp.maximum(jnp.linalg.norm(p['dir'], axis=0, keepdims=True), 1e-12)
    sdn = p['dir'] / dnorm
    theta = jax.nn.relu(ndn @ sdn)
    fout = fmap @ p['w'] + p['b']
    fc = fout[:, :, :out_ch]
    fs = fout[:, :, out_ch:]
    fs_nb = _gather_nb(fs, idx)
    act = (theta * fs_nb).reshape(bs, v, n, s, out_ch)
    act = jnp.max(act, axis=2)
    act = jnp.sum(act, axis=2)
    return fc + act


def _bn(p, x):
    mu = jnp.mean(x, axis=(0, 1), keepdims=True)
    var = jnp.var(x, axis=(0, 1), keepdims=True)
    return (x - mu) / jnp.sqrt(var + 1e-5) * p['g'] + p['b']


def _fusion_surface(p, vertices, knn, dim, s):
    il = knn[:, :, 1:_NB_L + 1]
    ig = knn[:, :, 1:_NB_G + 1]
    fl = jax.nn.relu(_bn(p['bn_l'], _conv_surface(p['conv_l'], il, vertices, dim, s)))
    fg = jax.nn.relu(_bn(p['bn_g0'], _conv_surface(p['conv_g0'], ig, vertices, dim, s)))
    fg = jax.nn.relu(_bn(p['bn_g1'], _conv_layer(p['conv_g1'], ig, vertices, fg, dim, s)))
    return jnp.concatenate([fl, fg], axis=2)


def _fusion(p, vertices, knn, inp, dim, s):
    il = knn[:, :, 1:_NB_L + 1]
    ig = knn[:, :, 1:_NB_G + 1]
    fl = jax.nn.relu(_bn(p['bn_l'], _conv_layer(p['conv_l'], il, vertices, inp, dim, s)))
    fg = jax.nn.relu(_bn(p['bn_g0'], _conv_layer(p['conv_g0'], ig, vertices, inp, dim, s)))
    fg = jax.nn.relu(_bn(p['bn_g1'], _conv_layer(p['conv_g1'], ig, vertices, fg, dim, s)))
    return jnp.concatenate([fl, fg], axis=2)


def _linear_relu(p, x):
    return jax.nn.relu(x @ p['w'] + p['b'])


def _pool(vertices, fmap, knn, rate=4, nbk=4):
    pool_num = vertices.shape[1] // rate
    idx = knn[:, :pool_num, 1:nbk + 1]
    nb = _gather_nb(fmap, idx)
    pooled = jnp.max(nb, axis=2)
    return vertices[:, :pool_num, :], pooled


def _transformer(p, pos, feat, knn, n_knn=16):
    identity = feat
    x = feat @ p['start']['w'] + p['start']['b']
    q = x @ p['q']['w'] + p['q']['b']
    k = x @ p['k']['w'] + p['k']['b']
    v = x @ p['v']['w'] + p['v']['b']
    idx = knn[:, :, 1:n_knn + 1]
    knb = _gather_nb(k, idx)
    vnb = _gather_nb(v, idx)
    pnb = _gather_nb_tc(pos, idx)
    rel = pos[:, :, None, :] - pnb
    pe = jax.nn.relu(rel @ p['pos1']['w'] + p['pos1']['b']) @ p['pos2']['w'] + p['pos2']['b']
    a = jax.nn.relu((q[:, :, None, :] - knb + pe) @ p['attn1']['w'] + p['attn1']['b']) @ p['attn2']['w'] + p['attn2']['b']
    a = jax.nn.softmax(a, axis=2)
    agg = jnp.sum(a * (vnb + pe), axis=2)
    return agg @ p['end']['w'] + p['end']['b'] + identity


def kernel(vertices, params):
    S = _SUPPORT
    v = jnp.transpose(vertices, (0, 2, 1))

    knn0 = _knn_topk(v, 51)
    fm0 = _fusion_surface(params['conv_0'], v, knn0, 128, S)
    fm0 = _linear_relu(params['down0'], fm0)
    fm0 = _transformer(params['att0'], v, fm0, knn0)
    fm1 = _fusion(params['conv_1'], v, knn0, fm0, 128, S)
    fm1 = _linear_relu(params['down1'], fm1)
    fm1 = _transformer(params['att1'], v, fm1, knn0)
    vp1, fp1 = _pool(v, fm1, knn0)

    knn1 = _knn_topk(vp1, 51)
    fm2 = _fusion(params['conv_2'], vp1, knn1, fp1, 128, S)
    fm2 = _transformer(params['att2'], vp1, fm2, knn1)
    fm3 = _fusion(params['conv_3'], vp1, knn1, fm2, 256, S)
    fm3 = _transformer(params['att3'], vp1, fm3, knn1)
    vp2, fp2 = _pool(vp1, fm3, knn1)

    knn2 = _knn_topk(vp2, 51)
    fm4 = _fusion(params['conv_4'], vp2, knn2, fp2, 512, S)
    fm4 = _linear_relu(params['down2'], fm4)
    fm4 = _transformer(params['att4'], vp2, fm4, knn2)
    return jnp.max(fm4, axis=1)
